# baseline (speedup 1.0000x reference)
import jax
import jax.numpy as jnp
import numpy as np
from jax.experimental import pallas as pl

N_IN = 64
N_OUT = 64
R = 8
D = 16
S = 25


OB = 32


def _plane_kernel(wt_ref, b_ref, q_ref, o_ref):
    rsub = jax.lax.broadcasted_iota(jnp.int32, (D * R, N_IN * R), 0)
    csub = jax.lax.broadcasted_iota(jnp.int32, (D * R, N_IN * R), 1)
    diag = (rsub % R) == (csub % R)
    wq_all = jnp.dot(wt_ref[...].reshape(OB * D, N_IN), q_ref[...],
                     preferred_element_type=jnp.float32)
    for ob in range(OB):
        wq = wq_all[ob * D:(ob + 1) * D]
        w_up = jnp.broadcast_to(wq[:, None, :], (D, R, N_IN * R))
        w_up = w_up.reshape(D * R, N_IN * R)
        rw = jnp.where(diag, w_up, 0.0)
        plane = jnp.dot(b_ref[...], rw,
                        preferred_element_type=jnp.float32)
        o_ref[:, ob] = plane.reshape(S, R, N_IN * R)


def kernel(weights, basis):
    wt = weights.reshape(N_OUT, N_IN, D).transpose(0, 2, 1)
    bmat = basis.transpose(3, 1, 0, 2).reshape(S * R, D * R)
    q = jnp.asarray(np.kron(np.eye(N_IN, dtype=np.float32),
                            np.ones((1, R), dtype=np.float32)))
    out = pl.pallas_call(
        _plane_kernel,
        grid=(N_OUT // OB,),
        in_specs=[
            pl.BlockSpec((OB, D, N_IN), lambda o: (o, 0, 0)),
            pl.BlockSpec((S * R, D * R), lambda o: (0, 0)),
            pl.BlockSpec((N_IN, N_IN * R), lambda o: (0, 0)),
        ],
        out_specs=pl.BlockSpec((S, OB, R, N_IN * R), lambda o: (0, o, 0, 0)),
        out_shape=jax.ShapeDtypeStruct((S, N_OUT, R, N_IN * R), jnp.float32),
    )(wt, bmat, q)
    return out.reshape(S, N_OUT * R, N_IN * R).transpose(1, 2, 0)


# speedup vs baseline: 1.0744x; 1.0744x over previous
import jax
import jax.numpy as jnp
import numpy as np
from jax.experimental import pallas as pl

N_IN = 64
N_OUT = 64
R = 8
D = 16
S = 25


OB = 16


def _plane_kernel(wt_ref, b_ref, q_ref, o_ref):
    rsub = jax.lax.broadcasted_iota(jnp.int32, (D * R, N_IN * R), 0)
    csub = jax.lax.broadcasted_iota(jnp.int32, (D * R, N_IN * R), 1)
    diag = (rsub % R) == (csub % R)
    wq_all = jnp.dot(wt_ref[...].reshape(OB * D, N_IN), q_ref[...],
                     preferred_element_type=jnp.float32)
    for ob in range(OB):
        wq = wq_all[ob * D:(ob + 1) * D]
        w_up = jnp.broadcast_to(wq[:, None, :], (D, R, N_IN * R))
        w_up = w_up.reshape(D * R, N_IN * R)
        rw = jnp.where(diag, w_up, 0.0)
        plane = jnp.dot(b_ref[...], rw,
                        preferred_element_type=jnp.float32)
        o_ref[:, ob] = plane.reshape(S, R, N_IN * R)


def kernel(weights, basis):
    wt = weights.reshape(N_OUT, N_IN, D).transpose(0, 2, 1)
    bmat = basis.transpose(3, 1, 0, 2).reshape(S * R, D * R)
    q = jnp.asarray(np.kron(np.eye(N_IN, dtype=np.float32),
                            np.ones((1, R), dtype=np.float32)))
    out = pl.pallas_call(
        _plane_kernel,
        grid=(N_OUT // OB,),
        in_specs=[
            pl.BlockSpec((OB, D, N_IN), lambda o: (o, 0, 0)),
            pl.BlockSpec((S * R, D * R), lambda o: (0, 0)),
            pl.BlockSpec((N_IN, N_IN * R), lambda o: (0, 0)),
        ],
        out_specs=pl.BlockSpec((S, OB, R, N_IN * R), lambda o: (0, o, 0, 0)),
        out_shape=jax.ShapeDtypeStruct((S, N_OUT, R, N_IN * R), jnp.float32),
    )(wt, bmat, q)
    return out.reshape(S, N_OUT * R, N_IN * R).transpose(1, 2, 0)
